# Initial kernel scaffold; baseline (speedup 1.0000x reference)
import jax
import jax.numpy as jnp
from jax.experimental import pallas as pl


def kernel(tokens, token_embedding):
    raise NotImplementedError("write your pallas kernel here")


# speedup vs baseline: 1.8700x; 1.8700x over previous
import functools

import jax
import jax.numpy as jnp
from jax import lax
from jax.experimental import pallas as pl
from jax.experimental.pallas import tpu as pltpu
from jax.experimental.pallas import tpu_sc as plsc

VOCAB = 1000000
DIM = 64
BATCH = 16384
HIST = 50
B = BATCH * HIST

NC, NS = 2, 16
NW = NC * NS
BPW = B // NW
CHUNK = 512
NCHUNK = BPW // CHUNK

_mesh = plsc.VectorSubcoreMesh(core_axis_name="c", subcore_axis_name="s")


@functools.partial(
    pl.kernel,
    out_type=jax.ShapeDtypeStruct((B, DIM), jnp.float32),
    mesh=_mesh,
    compiler_params=pltpu.CompilerParams(use_tc_tiling_on_sc=False),
    scratch_types=[
        pltpu.VMEM((BPW,), jnp.int32),
        pltpu.VMEM((2, CHUNK, DIM), jnp.float32),
        pltpu.SemaphoreType.DMA,
        pltpu.SemaphoreType.DMA,
    ],
)
def _gather_kernel(idx_hbm, table_hbm, out_hbm, idx_v, rows_v, gsem, osem):
    wid = lax.axis_index("s") * NC + lax.axis_index("c")
    base = wid * BPW
    pltpu.sync_copy(idx_hbm.at[pl.ds(base, BPW)], idx_v)

    def gather(i, slot):
        return pltpu.async_copy(
            table_hbm.at[idx_v.at[pl.ds(i * CHUNK, CHUNK)]],
            rows_v.at[slot],
            gsem,
        )

    def put(i, slot):
        return pltpu.async_copy(
            rows_v.at[slot],
            out_hbm.at[pl.ds(base + i * CHUNK, CHUNK)],
            osem,
        )

    gather(0, 0).wait()
    for i in range(1, NCHUNK):
        g = gather(i, i % 2)
        p = put(i - 1, (i - 1) % 2)
        g.wait()
        p.wait()
    put(NCHUNK - 1, (NCHUNK - 1) % 2).wait()


def kernel(tokens, token_embedding):
    flat = tokens.reshape(B)
    out = _gather_kernel(flat, token_embedding)
    return out.reshape(BATCH, HIST, DIM)


# speedup vs baseline: 2.8122x; 1.5039x over previous
import functools

import jax
import jax.numpy as jnp
from jax import lax
from jax.experimental import pallas as pl
from jax.experimental.pallas import tpu as pltpu
from jax.experimental.pallas import tpu_sc as plsc

VOCAB = 1000000
DIM = 64
BATCH = 16384
HIST = 50

NC, NS = 2, 16
DPC = DIM // NC
KMAX = 4
CB = BATCH // 2
NCB = BATCH // CB

_mesh = plsc.VectorSubcoreMesh(core_axis_name="c", subcore_axis_name="s")


@functools.partial(
    pl.kernel,
    out_type=jax.ShapeDtypeStruct((HIST, DIM, BATCH), jnp.float32),
    mesh=_mesh,
    scratch_types=[
        pltpu.VMEM_SHARED((VOCAB,), jnp.float32),
        pltpu.VMEM((2 * CB,), jnp.int32),
        pltpu.VMEM((2 * CB,), jnp.float32),
        pltpu.SemaphoreType.DMA,
        pltpu.SemaphoreType.DMA,
        pltpu.SemaphoreType.DMA,
    ],
)
def _embed_kernel(tok_hbm, table_hbm, out_hbm, row_sh, tokb, gbuf,
                  tsem, gsem, osem):
    c = lax.axis_index("c")
    s = lax.axis_index("s")
    nk = jnp.where(s + NS * (KMAX - 1) < HIST, KMAX, KMAX - 1)
    nu = nk * NCB

    def tok_src(u):
        k = u // NCB
        cb = u % NCB
        return tok_hbm.at[pl.ds((s + NS * k) * BATCH + cb * CB, CB)]

    def tok_dst(u):
        return tokb.at[pl.ds((u % 2) * CB, CB)]

    def gslot(u):
        return gbuf.at[pl.ds((u % 2) * CB, CB)]

    def out_dst(u, d):
        k = u // NCB
        cb = u % NCB
        return out_hbm.at[s + NS * k, d, pl.ds(cb * CB, CB)]

    def dstep(dloc, carry):
        d = c * DPC + dloc
        pltpu.async_copy(tok_src(0), tok_dst(0), tsem)
        plsc.subcore_barrier()

        @pl.when(s == 0)
        def _load_row():
            pltpu.sync_copy(table_hbm.at[d], row_sh)

        plsc.subcore_barrier()

        def ustep(u, carry):
            @pl.when(u + 1 < nu)
            def _prefetch_tok():
                pltpu.async_copy(tok_src(u + 1), tok_dst(u + 1), tsem)

            pltpu.make_async_copy(tok_src(u), tok_dst(u), tsem).wait()

            @pl.when(u >= 2)
            def _free_gslot():
                pltpu.make_async_copy(gslot(u), out_dst(u, d), osem).wait()

            pltpu.async_copy(row_sh.at[tok_dst(u)], gslot(u), gsem).wait()
            pltpu.async_copy(gslot(u), out_dst(u, d), osem)
            return carry

        lax.fori_loop(0, nu, ustep, 0)
        pltpu.make_async_copy(gslot(0), out_dst(0, d), osem).wait()
        pltpu.make_async_copy(gslot(1), out_dst(1, d), osem).wait()
        return carry

    lax.fori_loop(0, DPC, dstep, 0)


def kernel(tokens, token_embedding):
    tok_flat = tokens.T.reshape(HIST * BATCH)
    out_t = _embed_kernel(tok_flat, token_embedding.T)
    return jnp.transpose(out_t, (2, 0, 1))
